# baseline (device time: 45102 ns/iter reference)
import jax
import jax.numpy as jnp
from jax import lax
from jax.experimental import pallas as pl
from jax.experimental.pallas import tpu as pltpu

N_DEV = 4
N_STREAM = 4


def kernel(A, B):
    m, k = A.shape
    _, n = B.shape
    ch = m // N_DEV
    qw = n // (2 * N_STREAM)

    def body(a_ref, b_ref, out_ref, p_ref, rs_buf,
             rs_send, rs_recv, ag_send, ag_recv):
        my = lax.axis_index("i")
        left = (my - 1) % N_DEV
        right = (my + 1) % N_DEV

        barrier_sem = pltpu.get_barrier_semaphore()
        for nbr in [left, right]:
            pl.semaphore_signal(
                barrier_sem, inc=1,
                device_id=(nbr,), device_id_type=pl.DeviceIdType.MESH,
            )
        pl.semaphore_wait(barrier_sem, 2)

        def rows(c):
            return pl.ds((c % N_DEV) * ch, ch)

        def cols(d, q):
            return pl.ds((d * N_STREAM + q) * qw, qw)

        streams = [(d, q) for q in range(N_STREAM) for d in (0, 1)]

        def peer(d):
            return right if d == 0 else left

        def rs_send_chunk(d, s):
            return my - s if d == 0 else my + s

        def rs_recv_chunk(d, s):
            return my - s - 1 if d == 0 else my + s + 1

        def ag_send_chunk(d, s):
            return my + 1 - s if d == 0 else my - 1 + s

        def rs_rdma(d, q, s):
            src = (p_ref.at[rows(my), cols(d, q)] if s == 0
                   else rs_buf.at[d, q, s - 1])
            return pltpu.make_async_remote_copy(
                src_ref=src,
                dst_ref=rs_buf.at[d, q, s],
                send_sem=rs_send.at[d, q, s],
                recv_sem=rs_recv.at[d, q, s],
                device_id=(peer(d),),
                device_id_type=pl.DeviceIdType.MESH,
            )

        def ag_rdma(d, q, s):
            c = ag_send_chunk(d, s)
            return pltpu.make_async_remote_copy(
                src_ref=out_ref.at[rows(c), cols(d, q)],
                dst_ref=out_ref.at[rows(c), cols(d, q)],
                send_sem=ag_send.at[d, q, s],
                recv_sem=ag_recv.at[d, q, s],
                device_id=(peer(d),),
                device_id_type=pl.DeviceIdType.MESH,
            )

        rdmas = {}

        p_ref[rows(my), :] = jnp.dot(
            a_ref[rows(my), :], b_ref[...],
            preferred_element_type=jnp.float32)
        for d, q in streams:
            r = rdmas[("rs", d, q, 0)] = rs_rdma(d, q, 0)
            r.start()

        for j in (3, 1, 2):
            p_ref[rows(my + j), :] = jnp.dot(
                a_ref[rows(my + j), :], b_ref[...],
                preferred_element_type=jnp.float32)

        for s in range(1, N_DEV - 1):
            for d, q in streams:
                rdmas[("rs", d, q, s - 1)].wait_recv()
                rs_buf[d, q, s - 1] += p_ref[rows(rs_recv_chunk(d, s - 1)),
                                             cols(d, q)]
                r = rdmas[("rs", d, q, s)] = rs_rdma(d, q, s)
                r.start()

        s = N_DEV - 2
        for d, q in streams:
            rdmas[("rs", d, q, s)].wait_recv()
            c = rs_recv_chunk(d, s)
            out_ref[rows(c), cols(d, q)] = jnp.maximum(
                rs_buf[d, q, s] + p_ref[rows(c), cols(d, q)], 0.0)
            r = rdmas[("ag", d, q, 0)] = ag_rdma(d, q, 0)
            r.start()

        for s in range(1, N_DEV - 1):
            for d, q in streams:
                rdmas[("ag", d, q, s - 1)].wait_recv()
                r = rdmas[("ag", d, q, s)] = ag_rdma(d, q, s)
                r.start()
        for d, q in streams:
            rdmas[("ag", d, q, N_DEV - 2)].wait_recv()

        for key, r in rdmas.items():
            r.wait_send()

    return pl.pallas_call(
        body,
        out_shape=jax.ShapeDtypeStruct((m, n), jnp.float32),
        in_specs=[
            pl.BlockSpec(memory_space=pltpu.VMEM),
            pl.BlockSpec(memory_space=pltpu.VMEM),
        ],
        out_specs=pl.BlockSpec(memory_space=pltpu.VMEM),
        scratch_shapes=[
            pltpu.VMEM((m, n), jnp.float32),
            pltpu.VMEM((2, N_STREAM, N_DEV - 1, ch, qw), jnp.float32),
            pltpu.SemaphoreType.DMA((2, N_STREAM, N_DEV - 1)),
            pltpu.SemaphoreType.DMA((2, N_STREAM, N_DEV - 1)),
            pltpu.SemaphoreType.DMA((2, N_STREAM, N_DEV - 1)),
            pltpu.SemaphoreType.DMA((2, N_STREAM, N_DEV - 1)),
        ],
        compiler_params=pltpu.CompilerParams(collective_id=0),
    )(A, B)


# device time: 28918 ns/iter; 1.5597x vs baseline; 1.5597x over previous
import jax
import jax.numpy as jnp
from jax import lax
from jax.experimental import pallas as pl
from jax.experimental.pallas import tpu as pltpu

N_DEV = 4
N_STREAM = 2


def kernel(A, B):
    m, k = A.shape
    _, n = B.shape
    ch = m // N_DEV
    qw = n // (2 * N_STREAM)

    def body(a_ref, b_ref, out_ref, p_ref, rs_buf, ag_buf,
             rs_send, rs_recv, ag_send, ag_recv):
        my = lax.axis_index("i")
        left = (my - 1) % N_DEV
        right = (my + 1) % N_DEV

        barrier_sem = pltpu.get_barrier_semaphore()
        for nbr in [left, right]:
            pl.semaphore_signal(
                barrier_sem, inc=1,
                device_id=(nbr,), device_id_type=pl.DeviceIdType.MESH,
            )
        pl.semaphore_wait(barrier_sem, 2)

        def rows(c):
            return pl.ds((c % N_DEV) * ch, ch)

        def cols(d, q):
            return pl.ds((d * N_STREAM + q) * qw, qw)

        streams = [(d, q) for q in range(N_STREAM) for d in (0, 1)]

        def peer(d):
            return right if d == 0 else left

        def rs_recv_chunk(d, s):
            return my - s - 1 if d == 0 else my + s + 1

        def ag_recv_chunk(d, s):
            return my - s if d == 0 else my + s

        def rs_rdma(d, q, s):
            return pltpu.make_async_remote_copy(
                src_ref=rs_buf.at[d, q, s],
                dst_ref=rs_buf.at[d, q, s + 1],
                send_sem=rs_send.at[d, q, s],
                recv_sem=rs_recv.at[d, q, s],
                device_id=(peer(d),),
                device_id_type=pl.DeviceIdType.MESH,
            )

        def ag_rdma(d, q, s):
            return pltpu.make_async_remote_copy(
                src_ref=ag_buf.at[d, q, s],
                dst_ref=ag_buf.at[d, q, s + 1],
                send_sem=ag_send.at[d, q, s],
                recv_sem=ag_recv.at[d, q, s],
                device_id=(peer(d),),
                device_id_type=pl.DeviceIdType.MESH,
            )

        rdmas = {}

        p_ref[rows(my), :] = jnp.dot(
            a_ref[rows(my), :], b_ref[...],
            preferred_element_type=jnp.float32)
        for d, q in streams:
            rs_buf[d, q, 0] = p_ref[rows(my), cols(d, q)].astype(jnp.bfloat16)
            r = rdmas[("rs", d, q, 0)] = rs_rdma(d, q, 0)
            r.start()

        for j in (3, 1, 2):
            p_ref[rows(my + j), :] = jnp.dot(
                a_ref[rows(my + j), :], b_ref[...],
                preferred_element_type=jnp.float32)

        for s in range(1, N_DEV - 1):
            for d, q in streams:
                rdmas[("rs", d, q, s - 1)].wait_recv()
                c = rs_recv_chunk(d, s - 1)
                rs_buf[d, q, s] = (
                    rs_buf[d, q, s].astype(jnp.float32)
                    + p_ref[rows(c), cols(d, q)]
                ).astype(jnp.bfloat16)
                r = rdmas[("rs", d, q, s)] = rs_rdma(d, q, s)
                r.start()

        s = N_DEV - 2
        for d, q in streams:
            rdmas[("rs", d, q, s)].wait_recv()
            c = rs_recv_chunk(d, s)
            full = jnp.maximum(
                rs_buf[d, q, s + 1].astype(jnp.float32)
                + p_ref[rows(c), cols(d, q)], 0.0)
            ag_buf[d, q, 0] = full.astype(jnp.bfloat16)
            r = rdmas[("ag", d, q, 0)] = ag_rdma(d, q, 0)
            r.start()
            out_ref[rows(c), cols(d, q)] = full

        for s in range(1, N_DEV - 1):
            for d, q in streams:
                rdmas[("ag", d, q, s - 1)].wait_recv()
                r = rdmas[("ag", d, q, s)] = ag_rdma(d, q, s)
                r.start()
                c = ag_recv_chunk(d, s - 1)
                out_ref[rows(c), cols(d, q)] = (
                    ag_buf[d, q, s].astype(jnp.float32))
        for d, q in streams:
            s = N_DEV - 2
            rdmas[("ag", d, q, s)].wait_recv()
            c = ag_recv_chunk(d, s)
            out_ref[rows(c), cols(d, q)] = (
                ag_buf[d, q, s + 1].astype(jnp.float32))

        for r in rdmas.values():
            r.wait_send()

    return pl.pallas_call(
        body,
        out_shape=jax.ShapeDtypeStruct((m, n), jnp.float32),
        in_specs=[
            pl.BlockSpec(memory_space=pltpu.VMEM),
            pl.BlockSpec(memory_space=pltpu.VMEM),
        ],
        out_specs=pl.BlockSpec(memory_space=pltpu.VMEM),
        scratch_shapes=[
            pltpu.VMEM((m, n), jnp.float32),
            pltpu.VMEM((2, N_STREAM, N_DEV, ch, qw), jnp.bfloat16),
            pltpu.VMEM((2, N_STREAM, N_DEV, ch, qw), jnp.bfloat16),
            pltpu.SemaphoreType.DMA((2, N_STREAM, N_DEV - 1)),
            pltpu.SemaphoreType.DMA((2, N_STREAM, N_DEV - 1)),
            pltpu.SemaphoreType.DMA((2, N_STREAM, N_DEV - 1)),
            pltpu.SemaphoreType.DMA((2, N_STREAM, N_DEV - 1)),
        ],
        compiler_params=pltpu.CompilerParams(collective_id=0),
    )(A, B)
